# baseline (speedup 1.0000x reference)
import functools

import jax
import jax.numpy as jnp
from jax import lax
from jax.experimental import pallas as pl
from jax.experimental.pallas import tpu as pltpu
from jax.experimental.pallas import tpu_sc as plsc

D = 64
L = 20
NW = 32
NC = 2
GB = 32
SB = 16
LC = 10


def _rsqrt(x):
    i = plsc.bitcast(x, jnp.int32)
    i = jnp.int32(0x5F3759DF) - lax.shift_right_logical(i, jnp.int32(1))
    y = plsc.bitcast(i, jnp.float32)
    for _ in range(3):
        y = y * (jnp.float32(1.5) - jnp.float32(0.5) * x * y * y)
    return y


@functools.lru_cache(maxsize=None)
def _make_sc_kernel(B):
    NB = B // NW
    NG = NB // GB
    CROWS = GB * L
    mesh = plsc.VectorSubcoreMesh(core_axis_name="c", subcore_axis_name="s")

    @functools.partial(
        pl.kernel,
        out_type=jax.ShapeDtypeStruct((B * L,), jnp.float32),
        mesh=mesh,
        compiler_params=pltpu.CompilerParams(
            needs_layout_passes=False, use_tc_tiling_on_sc=False),
        scratch_types=[
            pltpu.VMEM((NB,), jnp.int32),
            pltpu.VMEM((NB * L,), jnp.int32),
            pltpu.VMEM((GB, D), jnp.float32),
            pltpu.VMEM((CROWS, D), jnp.float32),
            pltpu.VMEM((NB * L,), jnp.float32),
            pltpu.SemaphoreType.DMA,
        ],
    )
    def body(t_hbm, c_hbm, ti_hbm, ci_hbm, out_hbm, ti_v, ci_v, tr_v, cr_v,
             out_v, sem):
        wid = lax.axis_index("s") * NC + lax.axis_index("c")
        pltpu.sync_copy(ti_hbm.at[pl.ds(wid * NB, NB)], ti_v)
        pltpu.sync_copy(ci_hbm.at[pl.ds(wid * NB * L, NB * L)], ci_v)
        lanes = lax.iota(jnp.int32, 16)

        def group(g, carry):
            copies = [pltpu.async_copy(
                t_hbm.at[ti_v.at[pl.ds(g * GB, GB)]], tr_v, sem)]
            for k in range(CROWS // 128):
                copies.append(pltpu.async_copy(
                    c_hbm.at[ci_v.at[pl.ds(g * CROWS + k * 128, 128)]],
                    cr_v.at[pl.ds(k * 128, 128)], sem))
            for cp in copies:
                cp.wait()

            for sb in range(GB // SB):
                row16 = lanes + (sb * SB)
                ss_t = jnp.zeros((16,), jnp.float32)
                for d in range(D):
                    col = jnp.full((16,), d, jnp.int32)
                    tv = plsc.load_gather(tr_v, [row16, col])
                    ss_t = ss_t + tv * tv
                scale_t = jnp.minimum(jnp.float32(1.0), _rsqrt(ss_t))
                crow_base = row16 * L
                out_base = g * CROWS + row16 * L
                for lc in range(L // LC):

                    def dblk(dc, acc, lc=lc):
                        accd, accs = acc
                        col0 = dc * 16
                        tvs = []
                        for dd in range(16):
                            col = jnp.full((16,), col0 + dd, jnp.int32)
                            tvs.append(plsc.load_gather(tr_v, [row16, col]))
                        accd, accs = list(accd), list(accs)
                        for j in range(LC):
                            crow = crow_base + (lc * LC + j)
                            ad, asq = accd[j], accs[j]
                            for dd in range(16):
                                col = jnp.full((16,), col0 + dd, jnp.int32)
                                cv = plsc.load_gather(cr_v, [crow, col])
                                ad = ad + tvs[dd] * cv
                                asq = asq + cv * cv
                            accd[j], accs[j] = ad, asq
                        return tuple(accd), tuple(accs)

                    zd = tuple(jnp.zeros((16,), jnp.float32) for _ in range(LC))
                    zs = tuple(jnp.zeros((16,), jnp.float32) for _ in range(LC))
                    accd, accs = lax.fori_loop(0, D // 16, dblk, (zd, zs))
                    for j in range(LC):
                        scale_c = jnp.minimum(jnp.float32(1.0), _rsqrt(accs[j]))
                        val = accd[j] * scale_t * scale_c
                        plsc.store_scatter(out_v, [out_base + (lc * LC + j)], val)
            return carry

        lax.fori_loop(0, NG, group, 0)
        pltpu.sync_copy(out_v, out_hbm.at[pl.ds(wid * NB * L, NB * L)])

    return body


def kernel(inputs, context, t_table, c_table):
    B = inputs.shape[0]
    ti = inputs.astype(jnp.int32)
    ci = context.astype(jnp.int32).reshape(-1)
    out = _make_sc_kernel(B)(t_table, c_table, ti, ci)
    return out.reshape(B, L)
